# Initial kernel scaffold; baseline (speedup 1.0000x reference)
import jax
import jax.numpy as jnp
from jax.experimental import pallas as pl


def kernel(node_scalar, node_vector, adj_matrix, W1, b1, W2, b2, Wr, br):
    raise NotImplementedError("write your pallas kernel here")


# speedup vs baseline: 6.5020x; 6.5020x over previous
import functools

import jax
import jax.numpy as jnp
from jax import lax
from jax.experimental import pallas as pl
from jax.experimental.pallas import tpu as pltpu
from jax.experimental.pallas import tpu_sc as plsc

N_NODES = 10000
N_EDGES = 320000
F = 128
N_RBF = 20
CUTOFF = 5.0

NC = 2
NS = 16
CHUNK = 80
E_TILE = N_EDGES // NS
N_CHUNKS = E_TILE // CHUNK
N_TILE = N_NODES // NS

_BN = 400
_BE = 1000


def _node_tables_body(ns_ref, nvt_ref, w1_ref, b1_ref, w2_ref, b2_ref,
                      ts_ref, t0_ref, t1_ref, t2_ref):
    h = jnp.dot(ns_ref[...], w1_ref[...], preferred_element_type=jnp.float32)
    h = h + b1_ref[...]
    h = h * jax.nn.sigmoid(h)
    atom = jnp.dot(h, w2_ref[...], preferred_element_type=jnp.float32)
    atom = atom + b2_ref[...]
    a1 = atom[:, 0:F]
    a3 = atom[:, 2 * F:3 * F]
    ts_ref[...] = atom[:, F:2 * F]
    t0_ref[...] = jnp.concatenate([a1 * nvt_ref[:, 0:F], a3], axis=1)
    t1_ref[...] = jnp.concatenate([a1 * nvt_ref[:, F:2 * F], a3], axis=1)
    t2_ref[...] = jnp.concatenate([a1 * nvt_ref[:, 2 * F:3 * F], a3], axis=1)


def _node_tables(ns, nvt, W1, b1, W2, b2):
    grid = (N_NODES // _BN,)
    return pl.pallas_call(
        _node_tables_body,
        grid=grid,
        in_specs=[
            pl.BlockSpec((_BN, F), lambda i: (i, 0)),
            pl.BlockSpec((_BN, 3 * F), lambda i: (i, 0)),
            pl.BlockSpec((F, F), lambda i: (0, 0)),
            pl.BlockSpec((1, F), lambda i: (0, 0)),
            pl.BlockSpec((F, 3 * F), lambda i: (0, 0)),
            pl.BlockSpec((1, 3 * F), lambda i: (0, 0)),
        ],
        out_specs=[
            pl.BlockSpec((_BN, F), lambda i: (i, 0)),
            pl.BlockSpec((_BN, 2 * F), lambda i: (i, 0)),
            pl.BlockSpec((_BN, 2 * F), lambda i: (i, 0)),
            pl.BlockSpec((_BN, 2 * F), lambda i: (i, 0)),
        ],
        out_shape=[
            jax.ShapeDtypeStruct((N_NODES, F), jnp.float32),
            jax.ShapeDtypeStruct((N_NODES, 2 * F), jnp.float32),
            jax.ShapeDtypeStruct((N_NODES, 2 * F), jnp.float32),
            jax.ShapeDtypeStruct((N_NODES, 2 * F), jnp.float32),
        ],
    )(ns, nvt, W1, b1.reshape(1, F), W2, b2.reshape(1, 3 * F))


def _edge_streams_body(adj_ref, wr_ref, br_ref,
                       rb1_ref, rb2_ref, rb3_ref, rhat_ref):
    blk = adj_ref[...]
    d2 = blk[:, 5:6]
    nvals = lax.broadcasted_iota(jnp.int32, (1, N_RBF), 1).astype(
        jnp.float32) + 1.0
    arg = d2 * (nvals * (jnp.pi / CUTOFF))
    sinc = jnp.sin(arg) / d2
    rbf = jnp.dot(sinc, wr_ref[...], preferred_element_type=jnp.float32)
    rbf = rbf + br_ref[...]
    cc = jnp.where(d2 < CUTOFF,
                   0.5 * (jnp.cos(d2 * (jnp.pi / CUTOFF)) + 1.0), 0.0)
    rbcc = rbf * cc
    rb1_ref[...] = rbcc[:, 0:F]
    rb2_ref[...] = rbcc[:, F:2 * F]
    rb3_ref[...] = rbcc[:, 2 * F:3 * F]
    rhat = blk[:, 2:5] / d2
    rhat_ref[...] = jnp.concatenate(
        [rhat, jnp.zeros((rhat.shape[0], 1), jnp.float32)], axis=1)


def _edge_streams(adj8, Wr, br):
    grid = (N_EDGES // _BE,)
    return pl.pallas_call(
        _edge_streams_body,
        grid=grid,
        in_specs=[
            pl.BlockSpec((_BE, 8), lambda i: (i, 0)),
            pl.BlockSpec((N_RBF, 3 * F), lambda i: (0, 0)),
            pl.BlockSpec((1, 3 * F), lambda i: (0, 0)),
        ],
        out_specs=[
            pl.BlockSpec((_BE, F), lambda i: (i, 0)),
            pl.BlockSpec((_BE, F), lambda i: (i, 0)),
            pl.BlockSpec((_BE, F), lambda i: (i, 0)),
            pl.BlockSpec((_BE, 4), lambda i: (i, 0)),
        ],
        out_shape=[
            jax.ShapeDtypeStruct((N_EDGES, F), jnp.float32),
            jax.ShapeDtypeStruct((N_EDGES, F), jnp.float32),
            jax.ShapeDtypeStruct((N_EDGES, F), jnp.float32),
            jax.ShapeDtypeStruct((N_EDGES, 4), jnp.float32),
        ],
    )(adj8, Wr, br.reshape(1, 3 * F))


def _sc_body(idxi_hbm, idxj_hbm, ts_hbm, t0_hbm, t1_hbm, t2_hbm,
             rb1_hbm, rb2_hbm, rb3_hbm, rhat_hbm, init_hbm,
             out_hbm,
             acc, idxi_v, idxj_v, rows_v, rba_v,
             rhat_v, msg_v, gsem):
    ci = lax.axis_index("c")
    ti = lax.axis_index("s")
    nsl = pl.ds(ti * N_TILE, N_TILE)

    def do_pass(q):
        pltpu.sync_copy(init_hbm.at[pl.ds(q * N_NODES + ti * N_TILE, N_TILE)],
                        acc.at[nsl])
        plsc.subcore_barrier()

        def chunk_body(it, carry):
            e0 = ti * E_TILE + it * CHUNK
            esl = pl.ds(e0, CHUNK)
            pltpu.sync_copy(idxi_hbm.at[esl], idxi_v)
            pltpu.sync_copy(idxj_hbm.at[esl], idxj_v)
            if q == 0:
                pltpu.async_copy(ts_hbm.at[idxj_v], msg_v, gsem).wait()
                pltpu.sync_copy(rb2_hbm.at[esl], rba_v)

                def e_body(e, c2):
                    for fb in range(F // 16):
                        s = pl.ds(fb * 16, 16)
                        msg_v[e, s] = msg_v[e, s] * rba_v[e, s]
                    return c2
            else:
                table = (t0_hbm, t1_hbm, t2_hbm)[q - 1]
                pltpu.async_copy(table.at[idxj_v], rows_v, gsem).wait()
                pltpu.sync_copy(rb1_hbm.at[esl], rba_v)
                pltpu.sync_copy(rb3_hbm.at[esl], msg_v)
                pltpu.sync_copy(rhat_hbm.at[pl.ds(e0 * 4, CHUNK * 4)],
                                rhat_v.at[pl.ds(0, CHUNK * 4)])

                def e_body(e, c2):
                    rhvec = rhat_v[pl.ds(e * 4 + (q - 1), 16)]
                    rh = lax.broadcast(rhvec[0], (16,))
                    for fb in range(F // 16):
                        s = pl.ds(fb * 16, 16)
                        s2 = pl.ds(F + fb * 16, 16)
                        msg_v[e, s] = (rows_v[e, s] * rba_v[e, s]
                                       + rows_v[e, s2] * msg_v[e, s] * rh)
                    return c2

            lax.fori_loop(0, CHUNK, e_body, 0)
            pltpu.sync_copy(msg_v, acc.at[idxi_v], add=True)
            return carry

        lax.fori_loop(0, N_CHUNKS, chunk_body, 0)
        plsc.subcore_barrier()
        pltpu.sync_copy(acc.at[nsl],
                        out_hbm.at[pl.ds(q * N_NODES + ti * N_TILE, N_TILE)])
        plsc.subcore_barrier()

    @pl.when(ci == 0)
    def _():
        do_pass(0)
        do_pass(1)

    @pl.when(ci == 1)
    def _():
        do_pass(2)
        do_pass(3)


def _sc_scatter(idx_i, idx_j, ts, t0, t1, t2, rb1, rb2, rb3, rhat4, init_flat):
    mesh = plsc.VectorSubcoreMesh(
        core_axis_name="c", subcore_axis_name="s",
        num_cores=NC, num_subcores=NS)
    return pl.kernel(
        _sc_body,
        out_type=jax.ShapeDtypeStruct((4 * N_NODES, F), jnp.float32),
        mesh=mesh,
        compiler_params=pltpu.CompilerParams(use_tc_tiling_on_sc=False),
        scratch_types=[
            pltpu.VMEM_SHARED((N_NODES, F), jnp.float32),
            pltpu.VMEM((CHUNK,), jnp.int32),
            pltpu.VMEM((CHUNK,), jnp.int32),
            pltpu.VMEM((CHUNK, 2 * F), jnp.float32),
            pltpu.VMEM((CHUNK, F), jnp.float32),
            pltpu.VMEM((CHUNK * 4 + 16, ), jnp.float32),
            pltpu.VMEM((CHUNK, F), jnp.float32),
            pltpu.SemaphoreType.DMA,
        ],
    )(idx_i, idx_j, ts, t0, t1, t2, rb1, rb2, rb3, rhat4, init_flat)


@jax.jit
def kernel(node_scalar, node_vector, adj_matrix, W1, b1, W2, b2, Wr, br):
    idx_i = adj_matrix[:, 0].astype(jnp.int32)
    idx_j = adj_matrix[:, 1].astype(jnp.int32)
    nvt = node_vector.transpose(0, 2, 1).reshape(N_NODES, 3 * F)
    adj8 = jnp.concatenate(
        [adj_matrix, jnp.zeros((N_EDGES, 2), jnp.float32)], axis=1)

    ts, t0, t1, t2 = _node_tables(node_scalar, nvt, W1, b1, W2, b2)
    rb1, rb2, rb3, rhat4 = _edge_streams(adj8, Wr, br)
    init_flat = jnp.concatenate(
        [node_scalar, nvt[:, 0:F], nvt[:, F:2 * F], nvt[:, 2 * F:3 * F]],
        axis=0)

    out = _sc_scatter(idx_i, idx_j, ts, t0, t1, t2, rb1, rb2, rb3,
                      rhat4.reshape(-1), init_flat)
    out_s = out[0:N_NODES]
    out_v = jnp.stack(
        [out[N_NODES:2 * N_NODES], out[2 * N_NODES:3 * N_NODES],
         out[3 * N_NODES:4 * N_NODES]], axis=-1)
    return out_s, out_v


# speedup vs baseline: 9.4698x; 1.4564x over previous
import functools

import jax
import jax.numpy as jnp
from jax import lax
from jax.experimental import pallas as pl
from jax.experimental.pallas import tpu as pltpu
from jax.experimental.pallas import tpu_sc as plsc

N_NODES = 10000
N_EDGES = 320000
F = 128
N_RBF = 20
CUTOFF = 5.0

NC = 2
NS = 16
CHUNK = 40
E_TILE = N_EDGES // NS
N_CHUNKS = E_TILE // CHUNK
N_TILE = N_NODES // NS

_BN = 400
_BE = 1000


def _node_tables_body(ns_ref, nvt_ref, w1_ref, b1_ref, w2_ref, b2_ref,
                      ts_ref, t0_ref, t1_ref, t2_ref):
    h = jnp.dot(ns_ref[...], w1_ref[...], preferred_element_type=jnp.float32)
    h = h + b1_ref[...]
    h = h * jax.nn.sigmoid(h)
    atom = jnp.dot(h, w2_ref[...], preferred_element_type=jnp.float32)
    atom = atom + b2_ref[...]
    a1 = atom[:, 0:F]
    a3 = atom[:, 2 * F:3 * F]
    a2 = atom[:, F:2 * F]
    ts_ref[...] = jnp.concatenate([a2, a2], axis=1)
    t0_ref[...] = jnp.concatenate([a1 * nvt_ref[:, 0:F], a3], axis=1)
    t1_ref[...] = jnp.concatenate([a1 * nvt_ref[:, F:2 * F], a3], axis=1)
    t2_ref[...] = jnp.concatenate([a1 * nvt_ref[:, 2 * F:3 * F], a3], axis=1)


def _node_tables(ns, nvt, W1, b1, W2, b2):
    grid = (N_NODES // _BN,)
    return pl.pallas_call(
        _node_tables_body,
        grid=grid,
        in_specs=[
            pl.BlockSpec((_BN, F), lambda i: (i, 0)),
            pl.BlockSpec((_BN, 3 * F), lambda i: (i, 0)),
            pl.BlockSpec((F, F), lambda i: (0, 0)),
            pl.BlockSpec((1, F), lambda i: (0, 0)),
            pl.BlockSpec((F, 3 * F), lambda i: (0, 0)),
            pl.BlockSpec((1, 3 * F), lambda i: (0, 0)),
        ],
        out_specs=[
            pl.BlockSpec((_BN, 2 * F), lambda i: (i, 0)),
            pl.BlockSpec((_BN, 2 * F), lambda i: (i, 0)),
            pl.BlockSpec((_BN, 2 * F), lambda i: (i, 0)),
            pl.BlockSpec((_BN, 2 * F), lambda i: (i, 0)),
        ],
        out_shape=[
            jax.ShapeDtypeStruct((N_NODES, 2 * F), jnp.float32),
            jax.ShapeDtypeStruct((N_NODES, 2 * F), jnp.float32),
            jax.ShapeDtypeStruct((N_NODES, 2 * F), jnp.float32),
            jax.ShapeDtypeStruct((N_NODES, 2 * F), jnp.float32),
        ],
    )(ns, nvt, W1, b1.reshape(1, F), W2, b2.reshape(1, 3 * F))


def _edge_streams_body(adj_ref, wr_ref, br_ref,
                       rb2_ref, rb13_ref, rhat_ref):
    blk = adj_ref[...]
    d2 = blk[:, 5:6]
    nvals = lax.broadcasted_iota(jnp.int32, (1, N_RBF), 1).astype(
        jnp.float32) + 1.0
    arg = d2 * (nvals * (jnp.pi / CUTOFF))
    sinc = jnp.sin(arg) / d2
    rbf = jnp.dot(sinc, wr_ref[...], preferred_element_type=jnp.float32)
    rbf = rbf + br_ref[...]
    cc = jnp.where(d2 < CUTOFF,
                   0.5 * (jnp.cos(d2 * (jnp.pi / CUTOFF)) + 1.0), 0.0)
    rbcc = rbf * cc
    rb2_ref[...] = rbcc[:, F:2 * F]
    rb13_ref[...] = jnp.concatenate(
        [rbcc[:, 0:F], rbcc[:, 2 * F:3 * F]], axis=1)
    rhat = blk[:, 2:5] / d2
    rhat_ref[...] = jnp.concatenate(
        [rhat, jnp.zeros((rhat.shape[0], 1), jnp.float32)], axis=1)


def _edge_streams(adj8, Wr, br):
    grid = (N_EDGES // _BE,)
    return pl.pallas_call(
        _edge_streams_body,
        grid=grid,
        in_specs=[
            pl.BlockSpec((_BE, 8), lambda i: (i, 0)),
            pl.BlockSpec((N_RBF, 3 * F), lambda i: (0, 0)),
            pl.BlockSpec((1, 3 * F), lambda i: (0, 0)),
        ],
        out_specs=[
            pl.BlockSpec((_BE, F), lambda i: (i, 0)),
            pl.BlockSpec((_BE, 2 * F), lambda i: (i, 0)),
            pl.BlockSpec((_BE, 4), lambda i: (i, 0)),
        ],
        out_shape=[
            jax.ShapeDtypeStruct((N_EDGES, F), jnp.float32),
            jax.ShapeDtypeStruct((N_EDGES, 2 * F), jnp.float32),
            jax.ShapeDtypeStruct((N_EDGES, 4), jnp.float32),
        ],
    )(adj8, Wr, br.reshape(1, 3 * F))


def _sc_body(idxi_hbm, idxj_hbm, ts_hbm, t0_hbm, t1_hbm, t2_hbm,
             rb2_hbm, rb13_hbm, rhat_hbm, init_hbm,
             out_hbm,
             acc, rows_a, rows_b, rbx_a, rbx_b, msg_v,
             idxj_a, idxj_b, idxi_a, idxi_b, rhat_a, rhat_b,
             sem_a, sem_b, semj_a, semj_b):
    ci = lax.axis_index("c")
    ti = lax.axis_index("s")
    nsl = pl.ds(ti * N_TILE, N_TILE)
    bufs = ((rows_a, rbx_a, idxi_a, rhat_a, idxj_a, sem_a, semj_a),
            (rows_b, rbx_b, idxi_b, rhat_b, idxj_b, sem_b, semj_b))

    def do_pass(q):
        table = (ts_hbm, t0_hbm, t1_hbm, t2_hbm)[q]
        is_c = q != 0

        def e0_of(k):
            return ti * E_TILE + k * CHUNK

        def wrap(k):
            return lax.rem(k, N_CHUNKS)

        def start_batch(k, b):
            rows, rbx, idxi_v, rhat_v, idxj_v, sem, _ = bufs[b]
            e0 = e0_of(k)
            pltpu.async_copy(idxi_hbm.at[pl.ds(e0, CHUNK)], idxi_v, sem)
            if is_c:
                pltpu.async_copy(rb13_hbm.at[pl.ds(e0, CHUNK)], rbx, sem)
                pltpu.async_copy(rhat_hbm.at[pl.ds(e0 * 4, CHUNK * 4)],
                                 rhat_v.at[pl.ds(0, CHUNK * 4)], sem)
            pltpu.async_copy(table.at[idxj_v], rows, sem)

        def drain_batch(k, b):
            rows, rbx, idxi_v, rhat_v, idxj_v, sem, _ = bufs[b]
            e0 = e0_of(k)
            pltpu.make_async_copy(
                idxi_hbm.at[pl.ds(e0, CHUNK)], idxi_v, sem).wait()
            if is_c:
                pltpu.make_async_copy(
                    rb13_hbm.at[pl.ds(e0, CHUNK)], rbx, sem).wait()
                pltpu.make_async_copy(
                    rhat_hbm.at[pl.ds(e0 * 4, CHUNK * 4)],
                    rhat_v.at[pl.ds(0, CHUNK * 4)], sem).wait()
            pltpu.make_async_copy(table.at[idxj_v], rows, sem).wait()

        def start_idxj(k, b):
            _, _, _, _, idxj_v, _, semj = bufs[b]
            pltpu.async_copy(idxj_hbm.at[pl.ds(e0_of(k), CHUNK)], idxj_v, semj)

        def wait_idxj(k, b):
            _, _, _, _, idxj_v, _, semj = bufs[b]
            pltpu.make_async_copy(
                idxj_hbm.at[pl.ds(e0_of(k), CHUNK)], idxj_v, semj).wait()

        def compute_scatter(k, b):
            rows, rbx, idxi_v, rhat_v, _, _, _ = bufs[b]
            if not is_c:
                pltpu.sync_copy(rb2_hbm.at[pl.ds(e0_of(k), CHUNK)], msg_v)

                def e_body(e, c2):
                    for fb in range(F // 16):
                        s = pl.ds(fb * 16, 16)
                        msg_v[e, s] = msg_v[e, s] * rows[e, s]
                    return c2
            else:
                def e_body(e, c2):
                    rhvec = rhat_v[pl.ds(e * 4 + (q - 1), 16)]
                    rh = lax.broadcast(rhvec[0], (16,))
                    for fb in range(F // 16):
                        s = pl.ds(fb * 16, 16)
                        s2 = pl.ds(F + fb * 16, 16)
                        msg_v[e, s] = (rows[e, s] * rbx[e, s]
                                       + rows[e, s2] * rbx[e, s2] * rh)
                    return c2

            lax.fori_loop(0, CHUNK, e_body, 0)
            pltpu.sync_copy(msg_v, acc.at[idxi_v], add=True)

        pltpu.sync_copy(init_hbm.at[pl.ds(q * N_NODES + ti * N_TILE, N_TILE)],
                        acc.at[nsl])
        plsc.subcore_barrier()

        pltpu.sync_copy(idxj_hbm.at[pl.ds(e0_of(0), CHUNK)], idxj_a)
        start_batch(0, 0)
        start_idxj(1, 1)

        def loop_body(t, carry):
            for b in (0, 1):
                k = 2 * t + b
                drain_batch(k, b)
                wait_idxj(wrap(k + 1), 1 - b)
                start_batch(wrap(k + 1), 1 - b)
                start_idxj(wrap(k + 2), b)
                compute_scatter(k, b)
            return carry

        lax.fori_loop(0, N_CHUNKS // 2, loop_body, 0)
        drain_batch(0, 0)
        wait_idxj(1, 1)

        plsc.subcore_barrier()
        pltpu.sync_copy(acc.at[nsl],
                        out_hbm.at[pl.ds(q * N_NODES + ti * N_TILE, N_TILE)])
        plsc.subcore_barrier()

    @pl.when(ci == 0)
    def _():
        do_pass(0)
        do_pass(1)

    @pl.when(ci == 1)
    def _():
        do_pass(2)
        do_pass(3)


def _sc_scatter(idx_i, idx_j, ts, t0, t1, t2, rb2, rb13, rhat_flat, init_flat):
    mesh = plsc.VectorSubcoreMesh(
        core_axis_name="c", subcore_axis_name="s",
        num_cores=NC, num_subcores=NS)
    return pl.kernel(
        _sc_body,
        out_type=jax.ShapeDtypeStruct((4 * N_NODES, F), jnp.float32),
        mesh=mesh,
        compiler_params=pltpu.CompilerParams(use_tc_tiling_on_sc=False),
        scratch_types=[
            pltpu.VMEM_SHARED((N_NODES, F), jnp.float32),
            pltpu.VMEM((CHUNK, 2 * F), jnp.float32),
            pltpu.VMEM((CHUNK, 2 * F), jnp.float32),
            pltpu.VMEM((CHUNK, 2 * F), jnp.float32),
            pltpu.VMEM((CHUNK, 2 * F), jnp.float32),
            pltpu.VMEM((CHUNK, F), jnp.float32),
            pltpu.VMEM((CHUNK,), jnp.int32),
            pltpu.VMEM((CHUNK,), jnp.int32),
            pltpu.VMEM((CHUNK,), jnp.int32),
            pltpu.VMEM((CHUNK,), jnp.int32),
            pltpu.VMEM((CHUNK * 4 + 16,), jnp.float32),
            pltpu.VMEM((CHUNK * 4 + 16,), jnp.float32),
            pltpu.SemaphoreType.DMA,
            pltpu.SemaphoreType.DMA,
            pltpu.SemaphoreType.DMA,
            pltpu.SemaphoreType.DMA,
        ],
    )(idx_i, idx_j, ts, t0, t1, t2, rb2, rb13, rhat_flat, init_flat)


@jax.jit
def kernel(node_scalar, node_vector, adj_matrix, W1, b1, W2, b2, Wr, br):
    idx_i = adj_matrix[:, 0].astype(jnp.int32)
    idx_j = adj_matrix[:, 1].astype(jnp.int32)
    nvt = node_vector.transpose(0, 2, 1).reshape(N_NODES, 3 * F)
    adj8 = jnp.concatenate(
        [adj_matrix, jnp.zeros((N_EDGES, 2), jnp.float32)], axis=1)

    ts, t0, t1, t2 = _node_tables(node_scalar, nvt, W1, b1, W2, b2)
    rb2, rb13, rhat4 = _edge_streams(adj8, Wr, br)
    init_flat = jnp.concatenate(
        [node_scalar, nvt[:, 0:F], nvt[:, F:2 * F], nvt[:, 2 * F:3 * F]],
        axis=0)

    out = _sc_scatter(idx_i, idx_j, ts, t0, t1, t2, rb2, rb13,
                      rhat4.reshape(-1), init_flat)
    out_s = out[0:N_NODES]
    out_v = jnp.stack(
        [out[N_NODES:2 * N_NODES], out[2 * N_NODES:3 * N_NODES],
         out[3 * N_NODES:4 * N_NODES]], axis=-1)
    return out_s, out_v


# speedup vs baseline: 11.7226x; 1.2379x over previous
import functools

import jax
import jax.numpy as jnp
from jax import lax
from jax.experimental import pallas as pl
from jax.experimental.pallas import tpu as pltpu
from jax.experimental.pallas import tpu_sc as plsc

N_NODES = 10000
N_EDGES = 320000
F = 128
N_RBF = 20
CUTOFF = 5.0

NC = 2
NS = 16
CHUNK = 40
E_TILE = N_EDGES // NS
N_CHUNKS = E_TILE // CHUNK
N_TILE = N_NODES // NS

_BN = 400
_BE = 1280


def _node_tables_body(ns_ref, nvt_ref, w1_ref, b1_ref, w2_ref, b2_ref,
                      ts_ref, t0_ref, t1_ref, t2_ref):
    h = jnp.dot(ns_ref[...], w1_ref[...], preferred_element_type=jnp.float32)
    h = h + b1_ref[...]
    h = h * jax.nn.sigmoid(h)
    atom = jnp.dot(h, w2_ref[...], preferred_element_type=jnp.float32)
    atom = atom + b2_ref[...]
    a1 = atom[:, 0:F]
    a3 = atom[:, 2 * F:3 * F]
    a2 = atom[:, F:2 * F]
    ts_ref[...] = jnp.concatenate([a2, a2], axis=1)
    t0_ref[...] = jnp.concatenate([a1 * nvt_ref[:, 0:F], a3], axis=1)
    t1_ref[...] = jnp.concatenate([a1 * nvt_ref[:, F:2 * F], a3], axis=1)
    t2_ref[...] = jnp.concatenate([a1 * nvt_ref[:, 2 * F:3 * F], a3], axis=1)


def _node_tables(ns, nvt, W1, b1, W2, b2):
    grid = (N_NODES // _BN,)
    return pl.pallas_call(
        _node_tables_body,
        grid=grid,
        in_specs=[
            pl.BlockSpec((_BN, F), lambda i: (i, 0)),
            pl.BlockSpec((_BN, 3 * F), lambda i: (i, 0)),
            pl.BlockSpec((F, F), lambda i: (0, 0)),
            pl.BlockSpec((1, F), lambda i: (0, 0)),
            pl.BlockSpec((F, 3 * F), lambda i: (0, 0)),
            pl.BlockSpec((1, 3 * F), lambda i: (0, 0)),
        ],
        out_specs=[
            pl.BlockSpec((_BN, 2 * F), lambda i: (i, 0)),
            pl.BlockSpec((_BN, 2 * F), lambda i: (i, 0)),
            pl.BlockSpec((_BN, 2 * F), lambda i: (i, 0)),
            pl.BlockSpec((_BN, 2 * F), lambda i: (i, 0)),
        ],
        out_shape=[
            jax.ShapeDtypeStruct((N_NODES, 2 * F), jnp.float32),
            jax.ShapeDtypeStruct((N_NODES, 2 * F), jnp.float32),
            jax.ShapeDtypeStruct((N_NODES, 2 * F), jnp.float32),
            jax.ShapeDtypeStruct((N_NODES, 2 * F), jnp.float32),
        ],
    )(ns, nvt, W1, b1.reshape(1, F), W2, b2.reshape(1, 3 * F))


def _edge_streams_body(adj_ref, dflat_ref, wra_ref,
                       rb2_ref, rb13_ref, rhat_ref):
    blk = adj_ref[...]
    d2 = blk[:, 5:6]
    dx = dflat_ref[0, 0]
    x = dx * (jnp.pi / CUTOFF)
    s1 = jnp.sin(x)
    c2x = 2.0 * jnp.cos(x)
    cols = [s1, c2x * s1]
    for _k in range(2, N_RBF):
        cols.append(c2x * cols[-1] - cols[-2])
    cc = jnp.where(dx < CUTOFF, 0.5 * (jnp.cos(x) + 1.0), 0.0)
    ccd = cc / dx
    sinc_t = jnp.stack([c * ccd for c in cols] + [cc], axis=0)
    rbcc = jax.lax.dot_general(
        sinc_t, wra_ref[...], (((0,), (0,)), ((), ())),
        preferred_element_type=jnp.float32)
    rb2_ref[...] = rbcc[:, F:2 * F]
    rb13_ref[...] = jnp.concatenate(
        [rbcc[:, 0:F], rbcc[:, 2 * F:3 * F]], axis=1)
    rhat = blk[:, 2:5] / d2
    rhat_ref[...] = jnp.concatenate(
        [rhat, jnp.zeros((rhat.shape[0], 1), jnp.float32)], axis=1)


def _edge_streams(adj8, d_flat, Wr_aug):
    grid = (N_EDGES // _BE,)
    return pl.pallas_call(
        _edge_streams_body,
        grid=grid,
        in_specs=[
            pl.BlockSpec((_BE, 8), lambda i: (i, 0)),
            pl.BlockSpec((1, 1, _BE), lambda i: (i, 0, 0)),
            pl.BlockSpec((N_RBF + 1, 3 * F), lambda i: (0, 0)),
        ],
        out_specs=[
            pl.BlockSpec((_BE, F), lambda i: (i, 0)),
            pl.BlockSpec((_BE, 2 * F), lambda i: (i, 0)),
            pl.BlockSpec((_BE, 4), lambda i: (i, 0)),
        ],
        out_shape=[
            jax.ShapeDtypeStruct((N_EDGES, F), jnp.float32),
            jax.ShapeDtypeStruct((N_EDGES, 2 * F), jnp.float32),
            jax.ShapeDtypeStruct((N_EDGES, 4), jnp.float32),
        ],
    )(adj8, d_flat, Wr_aug)


def _sc_body(idxi_hbm, idxj_hbm, ts_hbm, t0_hbm, t1_hbm, t2_hbm,
             rb2_hbm, rb13_hbm, rhat_hbm, ns_hbm, nvt_hbm,
             out_hbm,
             acc, rows_a, rows_b, rbx_a, rbx_b, msg_v,
             idxj_a, idxj_b, idxi_a, idxi_b, rhat_a, rhat_b,
             sem_a, sem_b, semj_a, semj_b):
    ci = lax.axis_index("c")
    ti = lax.axis_index("s")
    nsl = pl.ds(ti * N_TILE, N_TILE)
    bufs = ((rows_a, rbx_a, idxi_a, rhat_a, idxj_a, sem_a, semj_a),
            (rows_b, rbx_b, idxi_b, rhat_b, idxj_b, sem_b, semj_b))

    def do_pass(q):
        table = (ts_hbm, t0_hbm, t1_hbm, t2_hbm)[q]
        is_c = q != 0

        def e0_of(k):
            return ti * E_TILE + k * CHUNK

        def wrap(k):
            return lax.rem(k, N_CHUNKS)

        def start_batch(k, b):
            rows, rbx, idxi_v, rhat_v, idxj_v, sem, _ = bufs[b]
            e0 = e0_of(k)
            pltpu.async_copy(idxi_hbm.at[pl.ds(e0, CHUNK)], idxi_v, sem)
            if is_c:
                pltpu.async_copy(rb13_hbm.at[pl.ds(e0, CHUNK)], rbx, sem)
                pltpu.async_copy(rhat_hbm.at[pl.ds(e0 * 4, CHUNK * 4)],
                                 rhat_v.at[pl.ds(0, CHUNK * 4)], sem)
            pltpu.async_copy(table.at[idxj_v], rows, sem)

        def drain_batch(k, b):
            rows, rbx, idxi_v, rhat_v, idxj_v, sem, _ = bufs[b]
            e0 = e0_of(k)
            pltpu.make_async_copy(
                idxi_hbm.at[pl.ds(e0, CHUNK)], idxi_v, sem).wait()
            if is_c:
                pltpu.make_async_copy(
                    rb13_hbm.at[pl.ds(e0, CHUNK)], rbx, sem).wait()
                pltpu.make_async_copy(
                    rhat_hbm.at[pl.ds(e0 * 4, CHUNK * 4)],
                    rhat_v.at[pl.ds(0, CHUNK * 4)], sem).wait()
            pltpu.make_async_copy(table.at[idxj_v], rows, sem).wait()

        def start_idxj(k, b):
            _, _, _, _, idxj_v, _, semj = bufs[b]
            pltpu.async_copy(idxj_hbm.at[pl.ds(e0_of(k), CHUNK)], idxj_v, semj)

        def wait_idxj(k, b):
            _, _, _, _, idxj_v, _, semj = bufs[b]
            pltpu.make_async_copy(
                idxj_hbm.at[pl.ds(e0_of(k), CHUNK)], idxj_v, semj).wait()

        def compute_scatter(k, b):
            rows, rbx, idxi_v, rhat_v, _, _, _ = bufs[b]
            if not is_c:
                pltpu.sync_copy(rb2_hbm.at[pl.ds(e0_of(k), CHUNK)], msg_v)

                def e_body(e, c2):
                    for fb in range(F // 16):
                        s = pl.ds(fb * 16, 16)
                        msg_v[e, s] = msg_v[e, s] * rows[e, s]
                    return c2
            else:
                def e_body(e, c2):
                    rhvec = rhat_v[pl.ds(e * 4 + (q - 1), 16)]
                    rh = lax.broadcast(rhvec[0], (16,))
                    for fb in range(F // 16):
                        s = pl.ds(fb * 16, 16)
                        s2 = pl.ds(F + fb * 16, 16)
                        msg_v[e, s] = (rows[e, s] * rbx[e, s]
                                       + rows[e, s2] * rbx[e, s2] * rh)
                    return c2

            lax.fori_loop(0, CHUNK, e_body, 0)
            pltpu.sync_copy(msg_v, acc.at[idxi_v], add=True)

        if q == 0:
            pltpu.sync_copy(ns_hbm.at[nsl], acc.at[nsl])
        else:
            pltpu.sync_copy(
                nvt_hbm.at[pl.ds((q - 1) * N_NODES + ti * N_TILE, N_TILE)],
                acc.at[nsl])
        plsc.subcore_barrier()

        pltpu.sync_copy(idxj_hbm.at[pl.ds(e0_of(0), CHUNK)], idxj_a)
        start_batch(0, 0)
        start_idxj(1, 1)

        def loop_body(t, carry):
            for b in (0, 1):
                k = 2 * t + b
                drain_batch(k, b)
                wait_idxj(wrap(k + 1), 1 - b)
                start_batch(wrap(k + 1), 1 - b)
                start_idxj(wrap(k + 2), b)
                compute_scatter(k, b)
            return carry

        lax.fori_loop(0, N_CHUNKS // 2, loop_body, 0)
        drain_batch(0, 0)
        wait_idxj(1, 1)

        plsc.subcore_barrier()
        pltpu.sync_copy(acc.at[nsl],
                        out_hbm.at[pl.ds(q * N_NODES + ti * N_TILE, N_TILE)])
        plsc.subcore_barrier()

    @pl.when(ci == 0)
    def _():
        do_pass(0)
        do_pass(1)

    @pl.when(ci == 1)
    def _():
        do_pass(2)
        do_pass(3)


def _sc_scatter(idx_i, idx_j, ts, t0, t1, t2, rb2, rb13, rhat_flat,
                ns, nvt_flat):
    mesh = plsc.VectorSubcoreMesh(
        core_axis_name="c", subcore_axis_name="s",
        num_cores=NC, num_subcores=NS)
    return pl.kernel(
        _sc_body,
        out_type=jax.ShapeDtypeStruct((4 * N_NODES, F), jnp.float32),
        mesh=mesh,
        compiler_params=pltpu.CompilerParams(use_tc_tiling_on_sc=False),
        scratch_types=[
            pltpu.VMEM_SHARED((N_NODES, F), jnp.float32),
            pltpu.VMEM((CHUNK, 2 * F), jnp.float32),
            pltpu.VMEM((CHUNK, 2 * F), jnp.float32),
            pltpu.VMEM((CHUNK, 2 * F), jnp.float32),
            pltpu.VMEM((CHUNK, 2 * F), jnp.float32),
            pltpu.VMEM((CHUNK, F), jnp.float32),
            pltpu.VMEM((CHUNK,), jnp.int32),
            pltpu.VMEM((CHUNK,), jnp.int32),
            pltpu.VMEM((CHUNK,), jnp.int32),
            pltpu.VMEM((CHUNK,), jnp.int32),
            pltpu.VMEM((CHUNK * 4 + 16,), jnp.float32),
            pltpu.VMEM((CHUNK * 4 + 16,), jnp.float32),
            pltpu.SemaphoreType.DMA,
            pltpu.SemaphoreType.DMA,
            pltpu.SemaphoreType.DMA,
            pltpu.SemaphoreType.DMA,
        ],
    )(idx_i, idx_j, ts, t0, t1, t2, rb2, rb13, rhat_flat, ns, nvt_flat)


@jax.jit
def kernel(node_scalar, node_vector, adj_matrix, W1, b1, W2, b2, Wr, br):
    idx_i = adj_matrix[:, 0].astype(jnp.int32)
    idx_j = adj_matrix[:, 1].astype(jnp.int32)
    nvt3 = node_vector.transpose(2, 0, 1)
    nvt = nvt3.transpose(1, 0, 2).reshape(N_NODES, 3 * F)
    adj8 = jnp.concatenate(
        [adj_matrix, jnp.zeros((N_EDGES, 2), jnp.float32)], axis=1)

    ts, t0, t1, t2 = _node_tables(node_scalar, nvt, W1, b1, W2, b2)
    wr_aug = jnp.concatenate([Wr, br.reshape(1, 3 * F)], axis=0)
    rb2, rb13, rhat4 = _edge_streams(
        adj8, adj_matrix[:, 5].reshape(N_EDGES // _BE, 1, _BE), wr_aug)

    out = _sc_scatter(idx_i, idx_j, ts, t0, t1, t2, rb2, rb13,
                      rhat4.reshape(-1), node_scalar,
                      nvt3.reshape(3 * N_NODES, F))
    out_s = out[0:N_NODES]
    out_v = jnp.stack(
        [out[N_NODES:2 * N_NODES], out[2 * N_NODES:3 * N_NODES],
         out[3 * N_NODES:4 * N_NODES]], axis=-1)
    return out_s, out_v


# speedup vs baseline: 16.8548x; 1.4378x over previous
import functools

import jax
import jax.numpy as jnp
from jax import lax
from jax.experimental import pallas as pl
from jax.experimental.pallas import tpu as pltpu
from jax.experimental.pallas import tpu_sc as plsc

N_NODES = 10000
N_EDGES = 320000
F = 128
N_RBF = 20
CUTOFF = 5.0

NC = 2
NS = 16
CHUNK = 40
E_TILE = N_EDGES // NS
N_CHUNKS = E_TILE // CHUNK
N_TILE = N_NODES // NS

_BN = 400
_BE = 1280


def _node_tables_body(ns_ref, nvt_ref, w1_ref, b1_ref, w2_ref, b2_ref,
                      ts_ref, t0_ref, t1_ref, t2_ref):
    h = jnp.dot(ns_ref[...], w1_ref[...], preferred_element_type=jnp.float32)
    h = h + b1_ref[...]
    h = h * jax.nn.sigmoid(h)
    atom = jnp.dot(h, w2_ref[...], preferred_element_type=jnp.float32)
    atom = atom + b2_ref[...]
    a1 = atom[:, 0:F]
    a3 = atom[:, 2 * F:3 * F]
    a2 = atom[:, F:2 * F]
    ts_ref[...] = jnp.concatenate([a2, a2], axis=1)
    t0_ref[...] = jnp.concatenate([a1 * nvt_ref[:, 0:F], a3], axis=1)
    t1_ref[...] = jnp.concatenate([a1 * nvt_ref[:, F:2 * F], a3], axis=1)
    t2_ref[...] = jnp.concatenate([a1 * nvt_ref[:, 2 * F:3 * F], a3], axis=1)


def _node_tables(ns, nvt, W1, b1, W2, b2):
    grid = (N_NODES // _BN,)
    return pl.pallas_call(
        _node_tables_body,
        grid=grid,
        in_specs=[
            pl.BlockSpec((_BN, F), lambda i: (i, 0)),
            pl.BlockSpec((_BN, 3 * F), lambda i: (i, 0)),
            pl.BlockSpec((F, F), lambda i: (0, 0)),
            pl.BlockSpec((1, F), lambda i: (0, 0)),
            pl.BlockSpec((F, 3 * F), lambda i: (0, 0)),
            pl.BlockSpec((1, 3 * F), lambda i: (0, 0)),
        ],
        out_specs=[
            pl.BlockSpec((_BN, 2 * F), lambda i: (i, 0)),
            pl.BlockSpec((_BN, 2 * F), lambda i: (i, 0)),
            pl.BlockSpec((_BN, 2 * F), lambda i: (i, 0)),
            pl.BlockSpec((_BN, 2 * F), lambda i: (i, 0)),
        ],
        out_shape=[
            jax.ShapeDtypeStruct((N_NODES, 2 * F), jnp.float32),
            jax.ShapeDtypeStruct((N_NODES, 2 * F), jnp.float32),
            jax.ShapeDtypeStruct((N_NODES, 2 * F), jnp.float32),
            jax.ShapeDtypeStruct((N_NODES, 2 * F), jnp.float32),
        ],
    )(ns, nvt, W1, b1.reshape(1, F), W2, b2.reshape(1, 3 * F))


def _edge_streams_body(adj_ref, dflat_ref, wra_ref,
                       rb2_ref, rb13_ref, rhat_ref):
    blk = adj_ref[...]
    d2 = blk[:, 5:6]
    dx = dflat_ref[0, 0]
    x = dx * (jnp.pi / CUTOFF)
    s1 = jnp.sin(x)
    c2x = 2.0 * jnp.cos(x)
    cols = [s1, c2x * s1]
    for _k in range(2, N_RBF):
        cols.append(c2x * cols[-1] - cols[-2])
    cc = jnp.where(dx < CUTOFF, 0.5 * (jnp.cos(x) + 1.0), 0.0)
    ccd = cc / dx
    sinc_t = jnp.stack([c * ccd for c in cols] + [cc], axis=0)
    rbcc = jax.lax.dot_general(
        sinc_t, wra_ref[...], (((0,), (0,)), ((), ())),
        preferred_element_type=jnp.float32)
    rb2_ref[...] = rbcc[:, F:2 * F]
    rb13_ref[...] = jnp.concatenate(
        [rbcc[:, 0:F], rbcc[:, 2 * F:3 * F]], axis=1)
    rhat = blk[:, 2:5] / d2
    rhat_ref[...] = jnp.concatenate(
        [rhat, jnp.zeros((rhat.shape[0], 1), jnp.float32)], axis=1)


def _edge_streams(adj8, d_flat, Wr_aug):
    grid = (N_EDGES // _BE,)
    return pl.pallas_call(
        _edge_streams_body,
        grid=grid,
        in_specs=[
            pl.BlockSpec((_BE, 8), lambda i: (i, 0)),
            pl.BlockSpec((1, 1, _BE), lambda i: (i, 0, 0)),
            pl.BlockSpec((N_RBF + 1, 3 * F), lambda i: (0, 0)),
        ],
        out_specs=[
            pl.BlockSpec((_BE, F), lambda i: (i, 0)),
            pl.BlockSpec((_BE, 2 * F), lambda i: (i, 0)),
            pl.BlockSpec((_BE, 4), lambda i: (i, 0)),
        ],
        out_shape=[
            jax.ShapeDtypeStruct((N_EDGES, F), jnp.float32),
            jax.ShapeDtypeStruct((N_EDGES, 2 * F), jnp.float32),
            jax.ShapeDtypeStruct((N_EDGES, 4), jnp.float32),
        ],
    )(adj8, d_flat, Wr_aug)


def _sc_body(idxi_hbm, idxj_hbm, ts_hbm, t0_hbm, t1_hbm, t2_hbm,
             rb2_hbm, rb13_hbm, rhat_hbm, ns_hbm, nvt_hbm,
             out_hbm,
             acc, rows_a, rows_b, rbx_a, rbx_b, msg_v,
             idxj_a, idxj_b, idxi_a, idxi_b, rhat_a, rhat_b,
             sem_a, sem_b, semj_a, semj_b):
    ci = lax.axis_index("c")
    ti = lax.axis_index("s")
    nsl = pl.ds(ti * N_TILE, N_TILE)
    bufs = ((rows_a, rbx_a, idxi_a, rhat_a, idxj_a, sem_a, semj_a),
            (rows_b, rbx_b, idxi_b, rhat_b, idxj_b, sem_b, semj_b))

    def do_pass(q):
        table = (ts_hbm, t0_hbm, t1_hbm, t2_hbm)[q]
        is_c = q != 0

        def e0_of(k):
            return ti * E_TILE + k * CHUNK

        def wrap(k):
            return lax.rem(k, N_CHUNKS)

        def start_batch(k, b):
            rows, rbx, idxi_v, rhat_v, idxj_v, sem, _ = bufs[b]
            e0 = e0_of(k)
            pltpu.async_copy(idxi_hbm.at[pl.ds(e0, CHUNK)], idxi_v, sem)
            if is_c:
                pltpu.async_copy(rb13_hbm.at[pl.ds(e0, CHUNK)], rbx, sem)
                pltpu.async_copy(rhat_hbm.at[pl.ds(e0 * 4, CHUNK * 4)],
                                 rhat_v.at[pl.ds(0, CHUNK * 4)], sem)
            pltpu.async_copy(table.at[idxj_v], rows, sem)

        def drain_batch(k, b):
            rows, rbx, idxi_v, rhat_v, idxj_v, sem, _ = bufs[b]
            e0 = e0_of(k)
            pltpu.make_async_copy(
                idxi_hbm.at[pl.ds(e0, CHUNK)], idxi_v, sem).wait()
            if is_c:
                pltpu.make_async_copy(
                    rb13_hbm.at[pl.ds(e0, CHUNK)], rbx, sem).wait()
                pltpu.make_async_copy(
                    rhat_hbm.at[pl.ds(e0 * 4, CHUNK * 4)],
                    rhat_v.at[pl.ds(0, CHUNK * 4)], sem).wait()
            pltpu.make_async_copy(table.at[idxj_v], rows, sem).wait()

        def start_idxj(k, b):
            _, _, _, _, idxj_v, _, semj = bufs[b]
            pltpu.async_copy(idxj_hbm.at[pl.ds(e0_of(k), CHUNK)], idxj_v, semj)

        def wait_idxj(k, b):
            _, _, _, _, idxj_v, _, semj = bufs[b]
            pltpu.make_async_copy(
                idxj_hbm.at[pl.ds(e0_of(k), CHUNK)], idxj_v, semj).wait()

        def compute_scatter(k, b):
            rows, rbx, idxi_v, rhat_v, _, _, _ = bufs[b]
            if not is_c:
                pltpu.sync_copy(rb2_hbm.at[pl.ds(e0_of(k), CHUNK)], msg_v)

                @plsc.parallel_loop(0, CHUNK, unroll=4)
                def _(e):
                    for fb in range(F // 16):
                        s = pl.ds(fb * 16, 16)
                        msg_v[e, s] = msg_v[e, s] * rows[e, s]
            else:
                @plsc.parallel_loop(0, CHUNK, unroll=4)
                def _(e):
                    rhvec = rhat_v[pl.ds(e * 4 + (q - 1), 16)]
                    rh = lax.broadcast(rhvec[0], (16,))
                    for fb in range(F // 16):
                        s = pl.ds(fb * 16, 16)
                        s2 = pl.ds(F + fb * 16, 16)
                        msg_v[e, s] = (rows[e, s] * rbx[e, s]
                                       + rows[e, s2] * rbx[e, s2] * rh)
            pltpu.sync_copy(msg_v, acc.at[idxi_v], add=True)

        if q == 0:
            pltpu.sync_copy(ns_hbm.at[nsl], acc.at[nsl])
        else:
            pltpu.sync_copy(
                nvt_hbm.at[pl.ds((q - 1) * N_NODES + ti * N_TILE, N_TILE)],
                acc.at[nsl])
        plsc.subcore_barrier()

        pltpu.sync_copy(idxj_hbm.at[pl.ds(e0_of(0), CHUNK)], idxj_a)
        start_batch(0, 0)
        start_idxj(1, 1)

        def loop_body(t, carry):
            for b in (0, 1):
                k = 2 * t + b
                drain_batch(k, b)
                wait_idxj(wrap(k + 1), 1 - b)
                start_batch(wrap(k + 1), 1 - b)
                start_idxj(wrap(k + 2), b)
                compute_scatter(k, b)
            return carry

        lax.fori_loop(0, N_CHUNKS // 2, loop_body, 0)
        drain_batch(0, 0)
        wait_idxj(1, 1)

        plsc.subcore_barrier()
        pltpu.sync_copy(acc.at[nsl],
                        out_hbm.at[pl.ds(q * N_NODES + ti * N_TILE, N_TILE)])
        plsc.subcore_barrier()

    @pl.when(ci == 0)
    def _():
        do_pass(0)
        do_pass(1)

    @pl.when(ci == 1)
    def _():
        do_pass(2)
        do_pass(3)


def _sc_scatter(idx_i, idx_j, ts, t0, t1, t2, rb2, rb13, rhat_flat,
                ns, nvt_flat):
    mesh = plsc.VectorSubcoreMesh(
        core_axis_name="c", subcore_axis_name="s",
        num_cores=NC, num_subcores=NS)
    return pl.kernel(
        _sc_body,
        out_type=jax.ShapeDtypeStruct((4 * N_NODES, F), jnp.float32),
        mesh=mesh,
        compiler_params=pltpu.CompilerParams(use_tc_tiling_on_sc=False),
        scratch_types=[
            pltpu.VMEM_SHARED((N_NODES, F), jnp.float32),
            pltpu.VMEM((CHUNK, 2 * F), jnp.float32),
            pltpu.VMEM((CHUNK, 2 * F), jnp.float32),
            pltpu.VMEM((CHUNK, 2 * F), jnp.float32),
            pltpu.VMEM((CHUNK, 2 * F), jnp.float32),
            pltpu.VMEM((CHUNK, F), jnp.float32),
            pltpu.VMEM((CHUNK,), jnp.int32),
            pltpu.VMEM((CHUNK,), jnp.int32),
            pltpu.VMEM((CHUNK,), jnp.int32),
            pltpu.VMEM((CHUNK,), jnp.int32),
            pltpu.VMEM((CHUNK * 4 + 16,), jnp.float32),
            pltpu.VMEM((CHUNK * 4 + 16,), jnp.float32),
            pltpu.SemaphoreType.DMA,
            pltpu.SemaphoreType.DMA,
            pltpu.SemaphoreType.DMA,
            pltpu.SemaphoreType.DMA,
        ],
    )(idx_i, idx_j, ts, t0, t1, t2, rb2, rb13, rhat_flat, ns, nvt_flat)


@jax.jit
def kernel(node_scalar, node_vector, adj_matrix, W1, b1, W2, b2, Wr, br):
    idx_i = adj_matrix[:, 0].astype(jnp.int32)
    idx_j = adj_matrix[:, 1].astype(jnp.int32)
    nvt3 = node_vector.transpose(2, 0, 1)
    nvt = nvt3.transpose(1, 0, 2).reshape(N_NODES, 3 * F)
    adj8 = jnp.concatenate(
        [adj_matrix, jnp.zeros((N_EDGES, 2), jnp.float32)], axis=1)

    ts, t0, t1, t2 = _node_tables(node_scalar, nvt, W1, b1, W2, b2)
    wr_aug = jnp.concatenate([Wr, br.reshape(1, 3 * F)], axis=0)
    rb2, rb13, rhat4 = _edge_streams(
        adj8, adj_matrix[:, 5].reshape(N_EDGES // _BE, 1, _BE), wr_aug)

    out = _sc_scatter(idx_i, idx_j, ts, t0, t1, t2, rb2, rb13,
                      rhat4.reshape(-1), node_scalar,
                      nvt3.reshape(3 * N_NODES, F))
    out_s = out[0:N_NODES]
    out_v = jnp.stack(
        [out[N_NODES:2 * N_NODES], out[2 * N_NODES:3 * N_NODES],
         out[3 * N_NODES:4 * N_NODES]], axis=-1)
    return out_s, out_v


# speedup vs baseline: 16.8661x; 1.0007x over previous
import functools

import jax
import jax.numpy as jnp
from jax import lax
from jax.experimental import pallas as pl
from jax.experimental.pallas import tpu as pltpu
from jax.experimental.pallas import tpu_sc as plsc

N_NODES = 10000
N_EDGES = 320000
F = 128
N_RBF = 20
CUTOFF = 5.0

NC = 2
NS = 16
CHUNK = 40
E_TILE = N_EDGES // NS
N_CHUNKS = E_TILE // CHUNK
N_TILE = N_NODES // NS

_BN = 400
_BE = 1280


def _node_tables_body(ns_ref, nvt_ref, w1_ref, b1_ref, w2_ref, b2_ref,
                      ts_ref, t0_ref, t1_ref, t2_ref):
    h = jnp.dot(ns_ref[...], w1_ref[...], preferred_element_type=jnp.float32)
    h = h + b1_ref[...]
    h = h * jax.nn.sigmoid(h)
    atom = jnp.dot(h, w2_ref[...], preferred_element_type=jnp.float32)
    atom = atom + b2_ref[...]
    a1 = atom[:, 0:F]
    a3 = atom[:, 2 * F:3 * F]
    a2 = atom[:, F:2 * F]
    ts_ref[...] = jnp.concatenate([a2, a2], axis=1)
    t0_ref[...] = jnp.concatenate([a1 * nvt_ref[:, 0:F], a3], axis=1)
    t1_ref[...] = jnp.concatenate([a1 * nvt_ref[:, F:2 * F], a3], axis=1)
    t2_ref[...] = jnp.concatenate([a1 * nvt_ref[:, 2 * F:3 * F], a3], axis=1)


def _node_tables(ns, nvt, W1, b1, W2, b2):
    grid = (N_NODES // _BN,)
    return pl.pallas_call(
        _node_tables_body,
        grid=grid,
        in_specs=[
            pl.BlockSpec((_BN, F), lambda i: (i, 0)),
            pl.BlockSpec((_BN, 3 * F), lambda i: (i, 0)),
            pl.BlockSpec((F, F), lambda i: (0, 0)),
            pl.BlockSpec((1, F), lambda i: (0, 0)),
            pl.BlockSpec((F, 3 * F), lambda i: (0, 0)),
            pl.BlockSpec((1, 3 * F), lambda i: (0, 0)),
        ],
        out_specs=[
            pl.BlockSpec((_BN, 2 * F), lambda i: (i, 0)),
            pl.BlockSpec((_BN, 2 * F), lambda i: (i, 0)),
            pl.BlockSpec((_BN, 2 * F), lambda i: (i, 0)),
            pl.BlockSpec((_BN, 2 * F), lambda i: (i, 0)),
        ],
        out_shape=[
            jax.ShapeDtypeStruct((N_NODES, 2 * F), jnp.float32),
            jax.ShapeDtypeStruct((N_NODES, 2 * F), jnp.float32),
            jax.ShapeDtypeStruct((N_NODES, 2 * F), jnp.float32),
            jax.ShapeDtypeStruct((N_NODES, 2 * F), jnp.float32),
        ],
    )(ns, nvt, W1, b1.reshape(1, F), W2, b2.reshape(1, 3 * F))


def _edge_streams_body(adj_ref, dflat_ref, wra_ref,
                       rb2_ref, rb13_ref, rhat_ref):
    blk = adj_ref[...]
    d2 = blk[:, 5:6]
    dx = dflat_ref[0, 0]
    x = dx * (jnp.pi / CUTOFF)
    s1 = jnp.sin(x)
    c2x = 2.0 * jnp.cos(x)
    cols = [s1, c2x * s1]
    for _k in range(2, N_RBF):
        cols.append(c2x * cols[-1] - cols[-2])
    cc = jnp.where(dx < CUTOFF, 0.5 * (jnp.cos(x) + 1.0), 0.0)
    ccd = cc / dx
    sinc_t = jnp.stack([c * ccd for c in cols] + [cc], axis=0)
    rbcc = jax.lax.dot_general(
        sinc_t, wra_ref[...], (((0,), (0,)), ((), ())),
        preferred_element_type=jnp.float32)
    rb2_ref[...] = rbcc[:, F:2 * F]
    rb13_ref[...] = jnp.concatenate(
        [rbcc[:, 0:F], rbcc[:, 2 * F:3 * F]], axis=1)
    rhat = blk[:, 2:5] / d2
    rhat_ref[...] = jnp.concatenate(
        [rhat, jnp.zeros((rhat.shape[0], 1), jnp.float32)], axis=1)


def _edge_streams(adj8, d_flat, Wr_aug):
    grid = (N_EDGES // _BE,)
    return pl.pallas_call(
        _edge_streams_body,
        grid=grid,
        in_specs=[
            pl.BlockSpec((_BE, 8), lambda i: (i, 0)),
            pl.BlockSpec((1, 1, _BE), lambda i: (i, 0, 0)),
            pl.BlockSpec((N_RBF + 1, 3 * F), lambda i: (0, 0)),
        ],
        out_specs=[
            pl.BlockSpec((_BE, F), lambda i: (i, 0)),
            pl.BlockSpec((_BE, 2 * F), lambda i: (i, 0)),
            pl.BlockSpec((_BE, 4), lambda i: (i, 0)),
        ],
        out_shape=[
            jax.ShapeDtypeStruct((N_EDGES, F), jnp.float32),
            jax.ShapeDtypeStruct((N_EDGES, 2 * F), jnp.float32),
            jax.ShapeDtypeStruct((N_EDGES, 4), jnp.float32),
        ],
    )(adj8, d_flat, Wr_aug)


def _sc_body(idxi_hbm, idxj_hbm, ts_hbm, t0_hbm, t1_hbm, t2_hbm,
             rb2_hbm, rb13_hbm, rhat_hbm, ns_hbm, nvt_hbm,
             out_hbm,
             acc, rows_a, rows_b, rbx_a, rbx_b, msg_v,
             idxj_a, idxj_b, idxi_a, idxi_b, rhat_a, rhat_b,
             sem_a, sem_b, semj_a, semj_b):
    ci = lax.axis_index("c")
    ti = lax.axis_index("s")
    nsl = pl.ds(ti * N_TILE, N_TILE)
    bufs = ((rows_a, rbx_a, idxi_a, rhat_a, idxj_a, sem_a, semj_a),
            (rows_b, rbx_b, idxi_b, rhat_b, idxj_b, sem_b, semj_b))

    def do_pass(q):
        table = (ts_hbm, t0_hbm, t1_hbm, t2_hbm)[q]
        is_c = q != 0

        def e0_of(k):
            return ti * E_TILE + k * CHUNK

        def wrap(k):
            return lax.rem(k, N_CHUNKS)

        def start_batch(k, b):
            rows, rbx, idxi_v, rhat_v, idxj_v, sem, _ = bufs[b]
            e0 = e0_of(k)
            pltpu.async_copy(idxi_hbm.at[pl.ds(e0, CHUNK)], idxi_v, sem)
            if is_c:
                pltpu.async_copy(rb13_hbm.at[pl.ds(e0, CHUNK)], rbx, sem)
                pltpu.async_copy(rhat_hbm.at[pl.ds(e0 * 4, CHUNK * 4)],
                                 rhat_v.at[pl.ds(0, CHUNK * 4)], sem)
            pltpu.async_copy(table.at[idxj_v], rows, sem)

        def drain_batch(k, b):
            rows, rbx, idxi_v, rhat_v, idxj_v, sem, _ = bufs[b]
            e0 = e0_of(k)
            pltpu.make_async_copy(
                idxi_hbm.at[pl.ds(e0, CHUNK)], idxi_v, sem).wait()
            if is_c:
                pltpu.make_async_copy(
                    rb13_hbm.at[pl.ds(e0, CHUNK)], rbx, sem).wait()
                pltpu.make_async_copy(
                    rhat_hbm.at[pl.ds(e0 * 4, CHUNK * 4)],
                    rhat_v.at[pl.ds(0, CHUNK * 4)], sem).wait()
            pltpu.make_async_copy(table.at[idxj_v], rows, sem).wait()

        def start_idxj(k, b):
            _, _, _, _, idxj_v, _, semj = bufs[b]
            pltpu.async_copy(idxj_hbm.at[pl.ds(e0_of(k), CHUNK)], idxj_v, semj)

        def wait_idxj(k, b):
            _, _, _, _, idxj_v, _, semj = bufs[b]
            pltpu.make_async_copy(
                idxj_hbm.at[pl.ds(e0_of(k), CHUNK)], idxj_v, semj).wait()

        def compute_scatter(k, b):
            rows, rbx, idxi_v, rhat_v, _, _, _ = bufs[b]
            if not is_c:
                pltpu.sync_copy(rb2_hbm.at[pl.ds(e0_of(k), CHUNK)], msg_v)

                @plsc.parallel_loop(0, CHUNK, unroll=8)
                def _(e):
                    for fb in range(F // 16):
                        s = pl.ds(fb * 16, 16)
                        msg_v[e, s] = msg_v[e, s] * rows[e, s]
            else:
                @plsc.parallel_loop(0, CHUNK, unroll=8)
                def _(e):
                    rhvec = rhat_v[pl.ds(e * 4 + (q - 1), 16)]
                    rh = lax.broadcast(rhvec[0], (16,))
                    for fb in range(F // 16):
                        s = pl.ds(fb * 16, 16)
                        s2 = pl.ds(F + fb * 16, 16)
                        msg_v[e, s] = (rows[e, s] * rbx[e, s]
                                       + rows[e, s2] * rbx[e, s2] * rh)
            pltpu.sync_copy(msg_v, acc.at[idxi_v], add=True)

        if q == 0:
            pltpu.sync_copy(ns_hbm.at[nsl], acc.at[nsl])
        else:
            pltpu.sync_copy(
                nvt_hbm.at[pl.ds((q - 1) * N_NODES + ti * N_TILE, N_TILE)],
                acc.at[nsl])
        plsc.subcore_barrier()

        pltpu.sync_copy(idxj_hbm.at[pl.ds(e0_of(0), CHUNK)], idxj_a)
        start_batch(0, 0)
        start_idxj(1, 1)

        def loop_body(t, carry):
            for b in (0, 1):
                k = 2 * t + b
                drain_batch(k, b)
                wait_idxj(wrap(k + 1), 1 - b)
                start_batch(wrap(k + 1), 1 - b)
                start_idxj(wrap(k + 2), b)
                compute_scatter(k, b)
            return carry

        lax.fori_loop(0, N_CHUNKS // 2, loop_body, 0)
        drain_batch(0, 0)
        wait_idxj(1, 1)

        plsc.subcore_barrier()
        pltpu.sync_copy(acc.at[nsl],
                        out_hbm.at[pl.ds(q * N_NODES + ti * N_TILE, N_TILE)])
        plsc.subcore_barrier()

    @pl.when(ci == 0)
    def _():
        do_pass(0)
        do_pass(1)

    @pl.when(ci == 1)
    def _():
        do_pass(2)
        do_pass(3)


def _sc_scatter(idx_i, idx_j, ts, t0, t1, t2, rb2, rb13, rhat_flat,
                ns, nvt_flat):
    mesh = plsc.VectorSubcoreMesh(
        core_axis_name="c", subcore_axis_name="s",
        num_cores=NC, num_subcores=NS)
    return pl.kernel(
        _sc_body,
        out_type=jax.ShapeDtypeStruct((4 * N_NODES, F), jnp.float32),
        mesh=mesh,
        compiler_params=pltpu.CompilerParams(use_tc_tiling_on_sc=False),
        scratch_types=[
            pltpu.VMEM_SHARED((N_NODES, F), jnp.float32),
            pltpu.VMEM((CHUNK, 2 * F), jnp.float32),
            pltpu.VMEM((CHUNK, 2 * F), jnp.float32),
            pltpu.VMEM((CHUNK, 2 * F), jnp.float32),
            pltpu.VMEM((CHUNK, 2 * F), jnp.float32),
            pltpu.VMEM((CHUNK, F), jnp.float32),
            pltpu.VMEM((CHUNK,), jnp.int32),
            pltpu.VMEM((CHUNK,), jnp.int32),
            pltpu.VMEM((CHUNK,), jnp.int32),
            pltpu.VMEM((CHUNK,), jnp.int32),
            pltpu.VMEM((CHUNK * 4 + 16,), jnp.float32),
            pltpu.VMEM((CHUNK * 4 + 16,), jnp.float32),
            pltpu.SemaphoreType.DMA,
            pltpu.SemaphoreType.DMA,
            pltpu.SemaphoreType.DMA,
            pltpu.SemaphoreType.DMA,
        ],
    )(idx_i, idx_j, ts, t0, t1, t2, rb2, rb13, rhat_flat, ns, nvt_flat)


@jax.jit
def kernel(node_scalar, node_vector, adj_matrix, W1, b1, W2, b2, Wr, br):
    idx_i = adj_matrix[:, 0].astype(jnp.int32)
    idx_j = adj_matrix[:, 1].astype(jnp.int32)
    nvt3 = node_vector.transpose(2, 0, 1)
    nvt = nvt3.transpose(1, 0, 2).reshape(N_NODES, 3 * F)
    adj8 = jnp.concatenate(
        [adj_matrix, jnp.zeros((N_EDGES, 2), jnp.float32)], axis=1)

    ts, t0, t1, t2 = _node_tables(node_scalar, nvt, W1, b1, W2, b2)
    wr_aug = jnp.concatenate([Wr, br.reshape(1, 3 * F)], axis=0)
    rb2, rb13, rhat4 = _edge_streams(
        adj8, adj_matrix[:, 5].reshape(N_EDGES // _BE, 1, _BE), wr_aug)

    out = _sc_scatter(idx_i, idx_j, ts, t0, t1, t2, rb2, rb13,
                      rhat4.reshape(-1), node_scalar,
                      nvt3.reshape(3 * N_NODES, F))
    out_s = out[0:N_NODES]
    out_v = jnp.stack(
        [out[N_NODES:2 * N_NODES], out[2 * N_NODES:3 * N_NODES],
         out[3 * N_NODES:4 * N_NODES]], axis=-1)
    return out_s, out_v
